# baseline (speedup 1.0000x reference)
import functools

import jax
import jax.numpy as jnp
from jax import lax
from jax.experimental import pallas as pl
from jax.experimental.pallas import tpu as pltpu
from jax.experimental.pallas import tpu_sc as plsc

NUM_EMB = 1000000
DIM = 32
BATCH = 16384

NC = 2
NS = 16
NW = NC * NS
BPW = BATCH // NW
IDX_MINOR = 128
NJ = BPW // IDX_MINOR
NCH = BPW // 16

TV_PAD = 1 << 20
A_BLK = 65536
A_GRID = TV_PAD // A_BLK
TVR = TV_PAD // 128


def _tv_body(wt_ref, tbl_ref, tv_ref):
    tv_ref[...] = jnp.dot(
        wt_ref[...], tbl_ref[...], preferred_element_type=jnp.float32
    ).reshape(A_BLK)


def _splat(vec16, lane):
    dnums = lax.GatherDimensionNumbers(
        offset_dims=(), collapsed_slice_dims=(0,), start_index_map=(0,)
    )
    idx = jnp.full((16, 1), lane, jnp.int32)
    return lax.gather(
        vec16, idx, dnums, slice_sizes=(1,),
        mode=lax.GatherScatterMode.PROMISE_IN_BOUNDS,
    )


def _sc_body(x_hbm, tv_hbm, b_hbm, out_hbm, xr_v, idx_v, rows_v, out_v, b_v,
             sem0, sem1, sem2, sem3):
    sems = (sem0, sem1, sem2, sem3)
    wid = lax.axis_index("s") * NC + lax.axis_index("c")
    base = wid * BPW

    pltpu.sync_copy(x_hbm.at[pl.ds(base, BPW)], xr_v)
    pltpu.sync_copy(b_hbm, b_v.at[pl.ds(0, 1)])

    def shift(c, carry):
        off = pl.multiple_of(c * 16, 16)
        idx_v[pl.ds(off, 16)] = lax.shift_right_logical(
            xr_v[pl.ds(off, 16)], 7
        )
        return carry

    lax.fori_loop(0, NCH, shift, 0)

    copies = [
        pltpu.async_copy(
            tv_hbm.at[idx_v.at[pl.ds(j * IDX_MINOR, IDX_MINOR)]],
            rows_v.at[pl.ds(j * IDX_MINOR, IDX_MINOR)],
            sems[j],
        )
        for j in range(NJ)
    ]
    bias = _splat(b_v[pl.ds(0, 16)], 0)
    lane_iota = lax.iota(jnp.int32, 16)

    def select(c, carry):
        coff = pl.multiple_of(c * 16, 16)
        xm = xr_v[pl.ds(coff, 16)] & 127
        woff_vec = jnp.minimum(xm & ~jnp.int32(7), 112)
        lane_vec = xm - woff_vec

        res = bias
        for j in range(16):
            woff = pl.multiple_of(woff_vec[j], 8)
            vals = rows_v[coff + j, pl.ds(woff, 16)]
            g = _splat(vals, lane_vec[j])
            res = jnp.where(lane_iota == j, g, res)
        out_v[pl.ds(coff, 16)] = 1.0 / (1.0 + jnp.exp(-(res + bias)))
        return carry

    for j in range(NJ):
        copies[j].wait()
        lax.fori_loop(j * (NCH // NJ), (j + 1) * (NCH // NJ), select, 0)

    pltpu.sync_copy(out_v, out_hbm.at[pl.ds(base, BPW)])


@jax.jit
def _run(x1, tableT, wt, b1):
    tv = pl.pallas_call(
        _tv_body,
        grid=(A_GRID,),
        in_specs=[
            pl.BlockSpec((1, DIM), lambda j: (0, 0)),
            pl.BlockSpec((DIM, A_BLK), lambda j: (0, j)),
        ],
        out_specs=pl.BlockSpec((A_BLK,), lambda j: (j,)),
        out_shape=jax.ShapeDtypeStruct((TV_PAD,), jnp.float32),
    )(wt, tableT)
    tv2 = tv.reshape(TVR, 128)

    mesh = plsc.VectorSubcoreMesh(core_axis_name="c", subcore_axis_name="s")
    fused = functools.partial(
        pl.kernel,
        mesh=mesh,
        out_type=jax.ShapeDtypeStruct((BATCH,), jnp.float32),
        scratch_types=[
            pltpu.VMEM((BPW,), jnp.int32),
            pltpu.VMEM((BPW,), jnp.int32),
            pltpu.VMEM((BPW, 128), jnp.float32),
            pltpu.VMEM((BPW,), jnp.float32),
            pltpu.VMEM((16,), jnp.float32),
            pltpu.SemaphoreType.DMA,
            pltpu.SemaphoreType.DMA,
            pltpu.SemaphoreType.DMA,
            pltpu.SemaphoreType.DMA,
        ],
    )(_sc_body)
    return fused(x1, tv2, b1)


def kernel(x, table, W, b):
    x1 = x.astype(jnp.int32)
    wt = W.reshape(1, DIM)
    return _run(x1, table.T, wt, b.reshape(1)).reshape(BATCH, 1)


# speedup vs baseline: 1.0080x; 1.0080x over previous
import functools

import jax
import jax.numpy as jnp
from jax import lax
from jax.experimental import pallas as pl
from jax.experimental.pallas import tpu as pltpu
from jax.experimental.pallas import tpu_sc as plsc

NUM_EMB = 1000000
DIM = 32
BATCH = 16384

NC = 2
NS = 16
NW = NC * NS
BPW = BATCH // NW
IDX_MINOR = 128
NJ = BPW // IDX_MINOR
NCH = BPW // 16

TV_PAD = 1 << 20
A_BLK = 65536
A_GRID = TV_PAD // A_BLK
TVR = TV_PAD // 128


def _tv_body(wt_ref, tbl_ref, tv_ref):
    tv_ref[...] = jnp.dot(
        wt_ref[...], tbl_ref[...], preferred_element_type=jnp.float32
    ).reshape(A_BLK)


def _splat(vec16, lane):
    dnums = lax.GatherDimensionNumbers(
        offset_dims=(), collapsed_slice_dims=(0,), start_index_map=(0,)
    )
    idx = jnp.full((16, 1), lane, jnp.int32)
    return lax.gather(
        vec16, idx, dnums, slice_sizes=(1,),
        mode=lax.GatherScatterMode.PROMISE_IN_BOUNDS,
    )


def _sc_body(x_hbm, tv_hbm, b_hbm, out_hbm, xr_v, idx_v, rows_v, out_v, b_v,
             sem0, sem1, sem2, sem3):
    sems = (sem0, sem1, sem2, sem3)
    wid = lax.axis_index("s") * NC + lax.axis_index("c")
    base = wid * BPW

    pltpu.sync_copy(x_hbm.at[pl.ds(base, BPW)], xr_v)
    pltpu.sync_copy(b_hbm, b_v.at[pl.ds(0, 1)])

    def shift(c, carry):
        off = pl.multiple_of(c * 16, 16)
        idx_v[pl.ds(off, 16)] = lax.shift_right_logical(
            xr_v[pl.ds(off, 16)], 7
        )
        return carry

    lax.fori_loop(0, NCH, shift, 0)

    copies = [
        pltpu.async_copy(
            tv_hbm.at[idx_v.at[pl.ds(j * IDX_MINOR, IDX_MINOR)]],
            rows_v.at[pl.ds(j * IDX_MINOR, IDX_MINOR)],
            sems[j],
        )
        for j in range(NJ)
    ]
    bias = _splat(b_v[pl.ds(0, 16)], 0)
    lane_iota = lax.iota(jnp.int32, 16)

    def select(c, carry):
        coff = pl.multiple_of(c * 16, 16)
        xm = xr_v[pl.ds(coff, 16)] & 127
        woff_vec = jnp.minimum(xm & ~jnp.int32(7), 112)
        lane_vec = xm - woff_vec

        res = bias
        for j in range(16):
            woff = pl.multiple_of(woff_vec[j], 8)
            vals = rows_v[coff + j, pl.ds(woff, 16)]
            g = _splat(vals, lane_vec[j])
            res = jnp.where(lane_iota == j, g, res)
        out_v[pl.ds(coff, 16)] = 1.0 / (1.0 + jnp.exp(-(res + bias)))
        return carry

    for c in copies:
        c.wait()
    lax.fori_loop(0, NCH, select, 0)

    pltpu.sync_copy(out_v, out_hbm.at[pl.ds(base, BPW)])


@jax.jit
def _run(x1, tableT, wt, b1):
    tv = pl.pallas_call(
        _tv_body,
        grid=(A_GRID,),
        in_specs=[
            pl.BlockSpec((1, DIM), lambda j: (0, 0)),
            pl.BlockSpec((DIM, A_BLK), lambda j: (0, j)),
        ],
        out_specs=pl.BlockSpec((A_BLK,), lambda j: (j,)),
        out_shape=jax.ShapeDtypeStruct((TV_PAD,), jnp.float32),
    )(wt, tableT)
    tv2 = tv.reshape(TVR, 128)

    mesh = plsc.VectorSubcoreMesh(core_axis_name="c", subcore_axis_name="s")
    fused = functools.partial(
        pl.kernel,
        mesh=mesh,
        out_type=jax.ShapeDtypeStruct((BATCH,), jnp.float32),
        scratch_types=[
            pltpu.VMEM((BPW,), jnp.int32),
            pltpu.VMEM((BPW,), jnp.int32),
            pltpu.VMEM((BPW, 128), jnp.float32),
            pltpu.VMEM((BPW,), jnp.float32),
            pltpu.VMEM((16,), jnp.float32),
            pltpu.SemaphoreType.DMA,
            pltpu.SemaphoreType.DMA,
            pltpu.SemaphoreType.DMA,
            pltpu.SemaphoreType.DMA,
        ],
    )(_sc_body)
    return fused(x1, tv2, b1)


def kernel(x, table, W, b):
    x1 = x.astype(jnp.int32)
    wt = W.reshape(1, DIM)
    return _run(x1, table.T, wt, b.reshape(1)).reshape(BATCH, 1)


# speedup vs baseline: 1.0083x; 1.0003x over previous
import functools

import jax
import jax.numpy as jnp
from jax import lax
from jax.experimental import pallas as pl
from jax.experimental.pallas import tpu as pltpu
from jax.experimental.pallas import tpu_sc as plsc

NUM_EMB = 1000000
DIM = 32
BATCH = 16384

NC = 2
NS = 16
NW = NC * NS
BPW = BATCH // NW
IDX_MINOR = 64
NJ = BPW // IDX_MINOR
NCH = BPW // 16

TV_PAD = 1 << 20
A_BLK = 65536
A_GRID = TV_PAD // A_BLK
TVR = TV_PAD // 128


def _tv_body(wt_ref, tbl_ref, tv_ref):
    tv_ref[...] = jnp.dot(
        wt_ref[...], tbl_ref[...], preferred_element_type=jnp.float32
    ).reshape(A_BLK)


def _splat(vec16, lane):
    dnums = lax.GatherDimensionNumbers(
        offset_dims=(), collapsed_slice_dims=(0,), start_index_map=(0,)
    )
    idx = jnp.full((16, 1), lane, jnp.int32)
    return lax.gather(
        vec16, idx, dnums, slice_sizes=(1,),
        mode=lax.GatherScatterMode.PROMISE_IN_BOUNDS,
    )


def _sc_body(x_hbm, tv_hbm, b_hbm, out_hbm, xr_v, idx_v, rows_v, out_v, b_v,
             sem0, sem1, sem2, sem3):
    sems = (sem0, sem1, sem2, sem3)
    wid = lax.axis_index("s") * NC + lax.axis_index("c")
    base = wid * BPW

    pltpu.sync_copy(x_hbm.at[pl.ds(base, BPW)], xr_v)
    pltpu.sync_copy(b_hbm, b_v.at[pl.ds(0, 1)])

    def shift(c, carry):
        off = pl.multiple_of(c * 16, 16)
        idx_v[pl.ds(off, 16)] = lax.shift_right_logical(
            xr_v[pl.ds(off, 16)], 7
        )
        return carry

    lax.fori_loop(0, NCH, shift, 0)

    copies = [
        pltpu.async_copy(
            tv_hbm.at[idx_v.at[pl.ds(j * IDX_MINOR, IDX_MINOR)]],
            rows_v.at[pl.ds(j * IDX_MINOR, IDX_MINOR)],
            sems[j % 4],
        )
        for j in range(NJ)
    ]
    bias = _splat(b_v[pl.ds(0, 16)], 0)
    lane_iota = lax.iota(jnp.int32, 16)

    def select(c, carry):
        coff = pl.multiple_of(c * 16, 16)
        xm = xr_v[pl.ds(coff, 16)] & 127
        woff_vec = jnp.minimum(xm & ~jnp.int32(7), 112)
        lane_vec = xm - woff_vec

        res = bias
        for j in range(16):
            woff = pl.multiple_of(woff_vec[j], 8)
            vals = rows_v[coff + j, pl.ds(woff, 16)]
            g = _splat(vals, lane_vec[j])
            res = jnp.where(lane_iota == j, g, res)
        out_v[pl.ds(coff, 16)] = 1.0 / (1.0 + jnp.exp(-(res + bias)))
        return carry

    for c in copies:
        c.wait()
    lax.fori_loop(0, NCH, select, 0)

    pltpu.sync_copy(out_v, out_hbm.at[pl.ds(base, BPW)])


@jax.jit
def _run(x1, tableT, wt, b1):
    tv = pl.pallas_call(
        _tv_body,
        grid=(A_GRID,),
        in_specs=[
            pl.BlockSpec((1, DIM), lambda j: (0, 0)),
            pl.BlockSpec((DIM, A_BLK), lambda j: (0, j)),
        ],
        out_specs=pl.BlockSpec((A_BLK,), lambda j: (j,)),
        out_shape=jax.ShapeDtypeStruct((TV_PAD,), jnp.float32),
    )(wt, tableT)
    tv2 = tv.reshape(TVR, 128)

    mesh = plsc.VectorSubcoreMesh(core_axis_name="c", subcore_axis_name="s")
    fused = functools.partial(
        pl.kernel,
        mesh=mesh,
        out_type=jax.ShapeDtypeStruct((BATCH,), jnp.float32),
        scratch_types=[
            pltpu.VMEM((BPW,), jnp.int32),
            pltpu.VMEM((BPW,), jnp.int32),
            pltpu.VMEM((BPW, 128), jnp.float32),
            pltpu.VMEM((BPW,), jnp.float32),
            pltpu.VMEM((16,), jnp.float32),
            pltpu.SemaphoreType.DMA,
            pltpu.SemaphoreType.DMA,
            pltpu.SemaphoreType.DMA,
            pltpu.SemaphoreType.DMA,
        ],
    )(_sc_body)
    return fused(x1, tv2, b1)


def kernel(x, table, W, b):
    x1 = x.astype(jnp.int32)
    wt = W.reshape(1, DIM)
    return _run(x1, table.T, wt, b.reshape(1)).reshape(BATCH, 1)
